# baseline (speedup 1.0000x reference)
import functools

import jax
import jax.numpy as jnp
from jax import lax
from jax.experimental import pallas as pl
from jax.experimental.pallas import tpu as pltpu
from jax.experimental.pallas import tpu_sc as plsc

N_ATOMS = 100000
N_PAIRS = 3200000
N_TYPES = 600

L = 16
PAD = 102400
NW = 32
ROWS_A = PAD // NW
TAIL = N_ATOMS - (NW - 1) * ROWS_A
PAIRS_PW = N_PAIRS // NW
C = 32
NCH = PAIRS_PW // C

_MESH = plsc.VectorSubcoreMesh(core_axis_name="c", subcore_axis_name="s")
_CP = pltpu.CompilerParams(needs_layout_passes=False,
                           use_tc_tiling_on_sc=False)


def _build_table(coords_cm, names, props):
    @functools.partial(
        pl.kernel,
        out_type=jax.ShapeDtypeStruct((PAD * 16,), jnp.float32),
        mesh=_MESH,
        compiler_params=_CP,
        scratch_types=[
            pltpu.VMEM((ROWS_A * 3,), jnp.float32),
            pltpu.VMEM((ROWS_A,), jnp.int32),
            pltpu.VMEM((N_TYPES,), jnp.float32),
            pltpu.VMEM((ROWS_A * 16,), jnp.float32),
        ],
    )
    def kern(coords_hbm, names_hbm, props_hbm, table_hbm, cbuf, nbuf, pbuf, tbuf):
        wid = lax.axis_index("c") * 16 + lax.axis_index("s")
        base = wid * ROWS_A

        @pl.when(wid < NW - 1)
        def _():
            for comp in range(3):
                pltpu.sync_copy(coords_hbm.at[comp, pl.ds(base, ROWS_A)],
                                cbuf.at[pl.ds(comp * ROWS_A, ROWS_A)])
            pltpu.sync_copy(names_hbm.at[pl.ds(base, ROWS_A)], nbuf)

        @pl.when(wid == NW - 1)
        def _():
            for comp in range(3):
                pltpu.sync_copy(coords_hbm.at[comp, pl.ds(base, TAIL)],
                                cbuf.at[pl.ds(comp * ROWS_A, TAIL)])
            pltpu.sync_copy(names_hbm.at[pl.ds(base, TAIL)], nbuf.at[pl.ds(0, TAIL)])

        pltpu.sync_copy(props_hbm, pbuf)

        iv = lax.iota(jnp.int32, L)
        c16 = jnp.full((L,), 16, jnp.int32)
        one = jnp.full((L,), 1, jnp.int32)
        two = jnp.full((L,), 2, jnp.int32)
        three = jnp.full((L,), 3, jnp.int32)
        nmax = jnp.full((L,), N_TYPES - 1, jnp.int32)

        def body(j, _):
            row = j * L + iv
            nm = jnp.minimum(nbuf[pl.ds(j * L, L)], nmax)
            r = plsc.load_gather(pbuf, [nm])
            x = cbuf[pl.ds(j * L, L)]
            y = cbuf[pl.ds(ROWS_A + j * L, L)]
            z = cbuf[pl.ds(2 * ROWS_A + j * L, L)]
            r16 = row * c16
            plsc.store_scatter(tbuf, [r16], x)
            plsc.store_scatter(tbuf, [r16 + one], y)
            plsc.store_scatter(tbuf, [r16 + two], z)
            plsc.store_scatter(tbuf, [r16 + three], r)
            return 0

        lax.fori_loop(0, ROWS_A // L, body, 0)
        pltpu.sync_copy(tbuf, table_hbm.at[pl.ds(base * 16, ROWS_A * 16)])

    return kern(coords_cm, names, props)


def _pair_sweep(table, pairs_flat, aux):
    C = 16
    BLK = 50
    NCHW = PAIRS_PW // C
    NBLK = NCHW // BLK
    BP = BLK * C
    NSEG = 5

    @functools.partial(
        pl.kernel,
        out_type=jax.ShapeDtypeStruct((NW, PAD // 128, 128), jnp.float32),
        mesh=_MESH,
        compiler_params=_CP,
        scratch_types=[
            pltpu.VMEM((PAD // 128, 128), jnp.float32),
            pltpu.VMEM((2 * BP,), jnp.int32),
            pltpu.VMEM((2 * BP,), jnp.int32),
            [pltpu.VMEM((C, 16), jnp.float32) for _ in range(2 * NSEG)],
            pltpu.VMEM((32,), jnp.float32),
            pltpu.SemaphoreType.DMA,
            pltpu.SemaphoreType.DMA,
            [pltpu.SemaphoreType.DMA for _ in range(NSEG)],
        ],
    )
    def kern(table_hbm, pairs_hbm, aux_hbm, stag_hbm,
             acc, px0, px1, ring, abuf, ps0, ps1, gsems):
        cid = lax.axis_index("c")
        sid = lax.axis_index("s")
        wid = cid * 16 + sid

        pltpu.sync_copy(aux_hbm, abuf)
        tol_v = abuf[pl.ds(0, L)]
        ew_v = jnp.exp(abuf[pl.ds(L, L)])

        zv = jnp.zeros((L,), jnp.float32)

        def zero_body(r, _):
            for cc in range(8):
                acc[r, pl.ds(cc * L, L)] = zv
            return 0

        lax.fori_loop(0, PAD // 128, zero_body, 0)

        iv = lax.iota(jnp.int32, L)
        k0 = jnp.zeros((L,), jnp.int32)
        k1 = jnp.full((L,), 1, jnp.int32)
        k2 = jnp.full((L,), 2, jnp.int32)
        k3 = jnp.full((L,), 3, jnp.int32)
        magic = jnp.full((L,), 0x5F3759DF, jnp.int32)
        one_i = jnp.full((L,), 1, jnp.int32)
        seven = jnp.full((L,), 7, jnp.int32)
        m127 = jnp.full((L,), 127, jnp.int32)
        half = jnp.full((L,), 0.5, jnp.float32)
        onep5 = jnp.full((L,), 1.5, jnp.float32)
        eps = jnp.full((L,), 1e-12, jnp.float32)

        pbufs = (px0, px1)
        psems = (ps0, ps1)
        base_off = wid * PAIRS_PW

        def pidx_start(pb, bk):
            off = base_off + bk * BP
            pltpu.async_copy(pairs_hbm.at[pl.ds(off, BP)],
                             pbufs[pb].at[pl.ds(0, BP)], psems[pb])
            pltpu.async_copy(pairs_hbm.at[pl.ds(N_PAIRS + off, BP)],
                             pbufs[pb].at[pl.ds(BP, BP)], psems[pb])

        def pidx_wait(pb):
            pltpu.make_async_copy(pairs_hbm.at[pl.ds(0, BP)],
                                  pbufs[pb].at[pl.ds(0, BP)],
                                  psems[pb]).wait()
            pltpu.make_async_copy(pairs_hbm.at[pl.ds(0, BP)],
                                  pbufs[pb].at[pl.ds(BP, BP)],
                                  psems[pb]).wait()

        def g_start(pb, seg, j):
            pb_ref = pbufs[pb]
            pltpu.async_copy(table_hbm.at[pb_ref.at[pl.ds(j * C, C)]],
                             ring[2 * seg], gsems[seg])
            pltpu.async_copy(table_hbm.at[pb_ref.at[pl.ds(BP + j * C, C)]],
                             ring[2 * seg + 1], gsems[seg])

        def g_wait(pb, seg, j):
            pb_ref = pbufs[pb]
            pltpu.make_async_copy(
                table_hbm.at[pb_ref.at[pl.ds(j * C, C)]],
                ring[2 * seg], gsems[seg]).wait()
            pltpu.make_async_copy(
                table_hbm.at[pb_ref.at[pl.ds(BP + j * C, C)]],
                ring[2 * seg + 1], gsems[seg]).wait()

        def compute(pb, seg, j):
            r0b = ring[2 * seg]
            r1b = ring[2 * seg + 1]
            pb_ref = pbufs[pb]
            e = iv
            a0 = pb_ref[pl.ds(j * C, L)]
            a1 = pb_ref[pl.ds(BP + j * C, L)]
            x0 = plsc.load_gather(r0b, [e, k0])
            y0 = plsc.load_gather(r0b, [e, k1])
            z0 = plsc.load_gather(r0b, [e, k2])
            r0 = plsc.load_gather(r0b, [e, k3])
            x1 = plsc.load_gather(r1b, [e, k0])
            y1 = plsc.load_gather(r1b, [e, k1])
            z1 = plsc.load_gather(r1b, [e, k2])
            r1 = plsc.load_gather(r1b, [e, k3])
            dx = x0 - x1
            dy = y0 - y1
            dz = z0 - z1
            d2 = dx * dx + dy * dy + dz * dz + eps
            ui = plsc.bitcast(d2, jnp.int32)
            ui = magic - lax.shift_right_logical(ui, one_i)
            yv = plsc.bitcast(ui, jnp.float32)
            hx = half * d2
            yv = yv * (onep5 - hx * yv * yv)
            yv = yv * (onep5 - hx * yv * yv)
            yv = yv * (onep5 - hx * yv * yv)
            dist = d2 * yv
            cc = r0 + r1 + tol_v
            ov = jnp.maximum(cc - dist, 0.0)
            cl = ov * ov * ew_v
            plsc.addupdate_scatter(
                acc, [lax.shift_right_logical(a0, seven),
                      jnp.bitwise_and(a0, m127)], cl)
            plsc.addupdate_scatter(
                acc, [lax.shift_right_logical(a1, seven),
                      jnp.bitwise_and(a1, m127)], cl)

        def block_body(pb, bk, cross):
            if cross:
                pidx_start(1 - pb, bk + 1)
            for j in range(BLK):
                seg = j % NSEG
                g_wait(pb, seg, j)
                if j < BLK - (NSEG - 1):
                    g_start(pb, (j + NSEG - 1) % NSEG, j + NSEG - 1)
                elif cross:
                    if j == BLK - (NSEG - 1):
                        pidx_wait(1 - pb)
                    g_start(1 - pb, (j + NSEG - 1) % NSEG,
                            j + NSEG - 1 - BLK)
                compute(pb, seg, j)

        pidx_start(0, 0)
        pidx_wait(0)
        for j in range(NSEG - 1):
            g_start(0, j, j)

        def outer(i, _):
            bk = i * 2
            block_body(0, bk, True)
            block_body(1, bk + 1, True)
            return 0

        lax.fori_loop(0, (NBLK - 1) // 2, outer, 0)
        block_body(0, NBLK - 1, False)

        pltpu.sync_copy(acc, stag_hbm.at[wid])

    return kern(table, pairs_flat, aux)


def _combine(partials):
    def add_kernel(a_ref, o_ref):
        o_ref[...] = jnp.sum(a_ref[...], axis=0)

    nrows = PAD // 128
    blk = 32

    out2d = pl.pallas_call(
        add_kernel,
        grid=(nrows // blk,),
        in_specs=[pl.BlockSpec((NW, blk, 128), lambda i: (0, i, 0))],
        out_specs=pl.BlockSpec((blk, 128), lambda i: (i, 0)),
        out_shape=jax.ShapeDtypeStruct((nrows, 128), jnp.float32),
    )(partials)
    return out2d.reshape(PAD)


def kernel(coords, atom_properties, tollerances, weight, atom_pairs, at_names):
    coords_cm = coords.T
    names = at_names.astype(jnp.int32)
    props = atom_properties[:, 0]
    pairs_flat = atom_pairs.T.reshape(-1).astype(jnp.int32)
    aux = jnp.concatenate([
        jnp.full((L,), tollerances[0], jnp.float32),
        jnp.full((L,), weight[0], jnp.float32),
    ])

    table = _build_table(coords_cm, names, props).reshape(PAD, 16)
    partials = _pair_sweep(table, pairs_flat, aux)
    return _combine(partials)[:N_ATOMS]


# speedup vs baseline: 1.5163x; 1.5163x over previous
import functools

import jax
import jax.numpy as jnp
from jax import lax
from jax.experimental import pallas as pl
from jax.experimental.pallas import tpu as pltpu
from jax.experimental.pallas import tpu_sc as plsc

N_ATOMS = 100000
N_PAIRS = 3200000
N_TYPES = 600

L = 16
PAD = 102400
NW = 32
ROWS_A = PAD // NW
TAIL = N_ATOMS - (NW - 1) * ROWS_A
PAIRS_PW = N_PAIRS // NW
C = 32
NCH = PAIRS_PW // C

_MESH = plsc.VectorSubcoreMesh(core_axis_name="c", subcore_axis_name="s")
_CP = pltpu.CompilerParams(needs_layout_passes=False,
                           use_tc_tiling_on_sc=False)


def _build_table(coords_cm, names, props):
    @functools.partial(
        pl.kernel,
        out_type=jax.ShapeDtypeStruct((PAD * 16,), jnp.float32),
        mesh=_MESH,
        compiler_params=_CP,
        scratch_types=[
            pltpu.VMEM((ROWS_A * 3,), jnp.float32),
            pltpu.VMEM((ROWS_A,), jnp.int32),
            pltpu.VMEM((N_TYPES,), jnp.float32),
            pltpu.VMEM((ROWS_A * 16,), jnp.float32),
        ],
    )
    def kern(coords_hbm, names_hbm, props_hbm, table_hbm, cbuf, nbuf, pbuf, tbuf):
        wid = lax.axis_index("c") * 16 + lax.axis_index("s")
        base = wid * ROWS_A

        @pl.when(wid < NW - 1)
        def _():
            for comp in range(3):
                pltpu.sync_copy(coords_hbm.at[comp, pl.ds(base, ROWS_A)],
                                cbuf.at[pl.ds(comp * ROWS_A, ROWS_A)])
            pltpu.sync_copy(names_hbm.at[pl.ds(base, ROWS_A)], nbuf)

        @pl.when(wid == NW - 1)
        def _():
            for comp in range(3):
                pltpu.sync_copy(coords_hbm.at[comp, pl.ds(base, TAIL)],
                                cbuf.at[pl.ds(comp * ROWS_A, TAIL)])
            pltpu.sync_copy(names_hbm.at[pl.ds(base, TAIL)], nbuf.at[pl.ds(0, TAIL)])

        pltpu.sync_copy(props_hbm, pbuf)

        iv = lax.iota(jnp.int32, L)
        c16 = jnp.full((L,), 16, jnp.int32)
        one = jnp.full((L,), 1, jnp.int32)
        two = jnp.full((L,), 2, jnp.int32)
        three = jnp.full((L,), 3, jnp.int32)
        nmax = jnp.full((L,), N_TYPES - 1, jnp.int32)

        def body(j, _):
            row = j * L + iv
            nm = jnp.minimum(nbuf[pl.ds(j * L, L)], nmax)
            r = plsc.load_gather(pbuf, [nm])
            x = cbuf[pl.ds(j * L, L)]
            y = cbuf[pl.ds(ROWS_A + j * L, L)]
            z = cbuf[pl.ds(2 * ROWS_A + j * L, L)]
            r16 = row * c16
            plsc.store_scatter(tbuf, [r16], x)
            plsc.store_scatter(tbuf, [r16 + one], y)
            plsc.store_scatter(tbuf, [r16 + two], z)
            plsc.store_scatter(tbuf, [r16 + three], r)
            return 0

        lax.fori_loop(0, ROWS_A // L, body, 0)
        pltpu.sync_copy(tbuf, table_hbm.at[pl.ds(base * 16, ROWS_A * 16)])

    return kern(coords_cm, names, props)


def _pair_sweep(table, pairs_flat, aux):
    C = 16
    BLK = 50
    NCHW = PAIRS_PW // C
    NBLK = NCHW // BLK
    BP = BLK * C
    NSEG = 10

    @functools.partial(
        pl.kernel,
        out_type=jax.ShapeDtypeStruct((NW, PAD // 128, 128), jnp.float32),
        mesh=_MESH,
        compiler_params=_CP,
        scratch_types=[
            pltpu.VMEM((PAD // 128, 128), jnp.float32),
            pltpu.VMEM((2 * BP,), jnp.int32),
            pltpu.VMEM((2 * BP,), jnp.int32),
            [pltpu.VMEM((C, 16), jnp.float32) for _ in range(2 * NSEG)],
            pltpu.VMEM((32,), jnp.float32),
            pltpu.SemaphoreType.DMA,
            pltpu.SemaphoreType.DMA,
            [pltpu.SemaphoreType.DMA for _ in range(NSEG)],
        ],
    )
    def kern(table_hbm, pairs_hbm, aux_hbm, stag_hbm,
             acc, px0, px1, ring, abuf, ps0, ps1, gsems):
        cid = lax.axis_index("c")
        sid = lax.axis_index("s")
        wid = cid * 16 + sid

        pltpu.sync_copy(aux_hbm, abuf)
        tol_v = abuf[pl.ds(0, L)]
        ew_v = jnp.exp(abuf[pl.ds(L, L)])

        zv = jnp.zeros((L,), jnp.float32)

        def zero_body(r, _):
            for cc in range(8):
                acc[r, pl.ds(cc * L, L)] = zv
            return 0

        lax.fori_loop(0, PAD // 128, zero_body, 0)

        iv = lax.iota(jnp.int32, L)
        k0 = jnp.zeros((L,), jnp.int32)
        k1 = jnp.full((L,), 1, jnp.int32)
        k2 = jnp.full((L,), 2, jnp.int32)
        k3 = jnp.full((L,), 3, jnp.int32)
        magic = jnp.full((L,), 0x5F3759DF, jnp.int32)
        one_i = jnp.full((L,), 1, jnp.int32)
        seven = jnp.full((L,), 7, jnp.int32)
        m127 = jnp.full((L,), 127, jnp.int32)
        half = jnp.full((L,), 0.5, jnp.float32)
        onep5 = jnp.full((L,), 1.5, jnp.float32)
        eps = jnp.full((L,), 1e-12, jnp.float32)

        pbufs = (px0, px1)
        psems = (ps0, ps1)
        base_off = wid * PAIRS_PW

        def pidx_start(pb, bk):
            off = base_off + bk * BP
            pltpu.async_copy(pairs_hbm.at[pl.ds(off, BP)],
                             pbufs[pb].at[pl.ds(0, BP)], psems[pb])
            pltpu.async_copy(pairs_hbm.at[pl.ds(N_PAIRS + off, BP)],
                             pbufs[pb].at[pl.ds(BP, BP)], psems[pb])

        def pidx_wait(pb):
            pltpu.make_async_copy(pairs_hbm.at[pl.ds(0, BP)],
                                  pbufs[pb].at[pl.ds(0, BP)],
                                  psems[pb]).wait()
            pltpu.make_async_copy(pairs_hbm.at[pl.ds(0, BP)],
                                  pbufs[pb].at[pl.ds(BP, BP)],
                                  psems[pb]).wait()

        def g_start(pb, seg, j):
            pb_ref = pbufs[pb]
            pltpu.async_copy(table_hbm.at[pb_ref.at[pl.ds(j * C, C)]],
                             ring[2 * seg], gsems[seg])
            pltpu.async_copy(table_hbm.at[pb_ref.at[pl.ds(BP + j * C, C)]],
                             ring[2 * seg + 1], gsems[seg])

        def g_wait(pb, seg, j):
            pb_ref = pbufs[pb]
            pltpu.make_async_copy(
                table_hbm.at[pb_ref.at[pl.ds(j * C, C)]],
                ring[2 * seg], gsems[seg]).wait()
            pltpu.make_async_copy(
                table_hbm.at[pb_ref.at[pl.ds(BP + j * C, C)]],
                ring[2 * seg + 1], gsems[seg]).wait()

        def compute(pb, seg, j):
            r0b = ring[2 * seg]
            r1b = ring[2 * seg + 1]
            pb_ref = pbufs[pb]
            e = iv
            a0 = pb_ref[pl.ds(j * C, L)]
            a1 = pb_ref[pl.ds(BP + j * C, L)]
            x0 = plsc.load_gather(r0b, [e, k0])
            y0 = plsc.load_gather(r0b, [e, k1])
            z0 = plsc.load_gather(r0b, [e, k2])
            r0 = plsc.load_gather(r0b, [e, k3])
            x1 = plsc.load_gather(r1b, [e, k0])
            y1 = plsc.load_gather(r1b, [e, k1])
            z1 = plsc.load_gather(r1b, [e, k2])
            r1 = plsc.load_gather(r1b, [e, k3])
            dx = x0 - x1
            dy = y0 - y1
            dz = z0 - z1
            d2 = dx * dx + dy * dy + dz * dz + eps
            ui = plsc.bitcast(d2, jnp.int32)
            ui = magic - lax.shift_right_logical(ui, one_i)
            yv = plsc.bitcast(ui, jnp.float32)
            hx = half * d2
            yv = yv * (onep5 - hx * yv * yv)
            yv = yv * (onep5 - hx * yv * yv)
            yv = yv * (onep5 - hx * yv * yv)
            dist = d2 * yv
            cc = r0 + r1 + tol_v
            ov = jnp.maximum(cc - dist, 0.0)
            cl = ov * ov * ew_v
            plsc.addupdate_scatter(
                acc, [lax.shift_right_logical(a0, seven),
                      jnp.bitwise_and(a0, m127)], cl)
            plsc.addupdate_scatter(
                acc, [lax.shift_right_logical(a1, seven),
                      jnp.bitwise_and(a1, m127)], cl)

        def block_body(pb, bk, cross):
            if cross:
                pidx_start(1 - pb, bk + 1)
            for j in range(BLK):
                seg = j % NSEG
                g_wait(pb, seg, j)
                if j < BLK - (NSEG - 1):
                    g_start(pb, (j + NSEG - 1) % NSEG, j + NSEG - 1)
                elif cross:
                    if j == BLK - (NSEG - 1):
                        pidx_wait(1 - pb)
                    g_start(1 - pb, (j + NSEG - 1) % NSEG,
                            j + NSEG - 1 - BLK)
                compute(pb, seg, j)

        pidx_start(0, 0)
        pidx_wait(0)
        for j in range(NSEG - 1):
            g_start(0, j, j)

        def outer(i, _):
            bk = i * 2
            block_body(0, bk, True)
            block_body(1, bk + 1, True)
            return 0

        lax.fori_loop(0, (NBLK - 1) // 2, outer, 0)
        block_body(0, NBLK - 1, False)

        pltpu.sync_copy(acc, stag_hbm.at[wid])

    return kern(table, pairs_flat, aux)


def _combine(partials):
    def add_kernel(a_ref, o_ref):
        o_ref[...] = jnp.sum(a_ref[...], axis=0)

    nrows = PAD // 128
    blk = 32

    out2d = pl.pallas_call(
        add_kernel,
        grid=(nrows // blk,),
        in_specs=[pl.BlockSpec((NW, blk, 128), lambda i: (0, i, 0))],
        out_specs=pl.BlockSpec((blk, 128), lambda i: (i, 0)),
        out_shape=jax.ShapeDtypeStruct((nrows, 128), jnp.float32),
    )(partials)
    return out2d.reshape(PAD)


def kernel(coords, atom_properties, tollerances, weight, atom_pairs, at_names):
    coords_cm = coords.T
    names = at_names.astype(jnp.int32)
    props = atom_properties[:, 0]
    pairs_flat = atom_pairs.T.reshape(-1).astype(jnp.int32)
    aux = jnp.concatenate([
        jnp.full((L,), tollerances[0], jnp.float32),
        jnp.full((L,), weight[0], jnp.float32),
    ])

    table = _build_table(coords_cm, names, props).reshape(PAD, 16)
    partials = _pair_sweep(table, pairs_flat, aux)
    return _combine(partials)[:N_ATOMS]
